# baseline (speedup 1.0000x reference)
import jax
import jax.numpy as jnp
from jax import lax
from jax.experimental import pallas as pl
from jax.experimental.pallas import tpu as pltpu
from jax.experimental.pallas import tpu_sc as plsc

B, L, V = 128, 2048, 33
LV = L * V

_INFO = plsc.get_sparse_core_info()
NC = _INFO.num_cores
NS = _INFO.num_subcores
LANES = _INFO.num_lanes
NW_ACTIVE = B // LANES


def _sc_body(flat_hbm, pos_hbm, mut_hbm, wt_hbm, out_hbm,
             pos_v, mut_v, wt_v, idx_v, gat_v, res_v, sem):
    wid = lax.axis_index("s") * NC + lax.axis_index("c")

    @pl.when(wid < NW_ACTIVE)
    def _():
        base = wid * LANES
        pltpu.sync_copy(pos_hbm.at[pl.ds(base, LANES)], pos_v)
        pltpu.sync_copy(mut_hbm.at[pl.ds(base, LANES)], mut_v)
        pltpu.sync_copy(wt_hbm.at[pl.ds(base, LANES)], wt_v)
        row = (base + lax.iota(jnp.int32, LANES)) * LV + pos_v[...] * V
        idx_v[pl.ds(0, LANES)] = row + mut_v[...]
        idx_v[pl.ds(LANES, LANES)] = row + wt_v[...]
        pltpu.async_copy(flat_hbm.at[idx_v], gat_v, sem).wait()
        res_v[...] = gat_v[pl.ds(0, LANES)] / gat_v[pl.ds(LANES, LANES)]
        pltpu.sync_copy(res_v, out_hbm.at[pl.ds(base, LANES)])


def kernel(wt_logits, mutated_position_idx, mutant_token_idx, wt_token_idx):
    flat = wt_logits.reshape(B * L * V)
    pos = mutated_position_idx.astype(jnp.int32)
    mut = mutant_token_idx.astype(jnp.int32)
    wt = wt_token_idx.astype(jnp.int32)
    mesh = plsc.VectorSubcoreMesh(core_axis_name="c", subcore_axis_name="s")
    run = pl.kernel(
        _sc_body,
        out_type=jax.ShapeDtypeStruct((B,), jnp.float32),
        mesh=mesh,
        scratch_types=[
            pltpu.VMEM((LANES,), jnp.int32),
            pltpu.VMEM((LANES,), jnp.int32),
            pltpu.VMEM((LANES,), jnp.int32),
            pltpu.VMEM((2 * LANES,), jnp.int32),
            pltpu.VMEM((2 * LANES,), jnp.float32),
            pltpu.VMEM((LANES,), jnp.float32),
            pltpu.SemaphoreType.DMA,
        ],
    )
    return run(flat, pos, mut, wt)


# speedup vs baseline: 2.2263x; 2.2263x over previous
import jax
import jax.numpy as jnp
from jax import lax
from jax.experimental import pallas as pl
from jax.experimental.pallas import tpu as pltpu
from jax.experimental.pallas import tpu_sc as plsc

B, L, V = 128, 2048, 33

_INFO = plsc.get_sparse_core_info()
NC = _INFO.num_cores
NS = _INFO.num_subcores
LANES = _INFO.num_lanes
NW_ACTIVE = B // LANES


def _pick(chunk, tok, lane_is_i, lo, hi, off, acc):
    idx = jnp.clip(tok - off, 0, LANES - 1)[:, None]
    g = jnp.take_along_axis(chunk, idx[:, 0], axis=0)
    cond = lane_is_i & (tok >= lo) & (tok < hi)
    return jnp.where(cond, g, acc)


def _sc_body(logits_hbm, pos_hbm, mut_hbm, wt_hbm, out_hbm,
             pos_v, mut_v, wt_v, rows_v, res_v, sem):
    wid = lax.axis_index("s") * NC + lax.axis_index("c")

    @pl.when(wid < NW_ACTIVE)
    def _():
        base = wid * LANES
        cp_p = pltpu.async_copy(pos_hbm.at[pl.ds(base, LANES)], pos_v, sem)
        cp_m = pltpu.async_copy(mut_hbm.at[pl.ds(base, LANES)], mut_v, sem)
        cp_w = pltpu.async_copy(wt_hbm.at[pl.ds(base, LANES)], wt_v, sem)
        cp_p.wait()
        cp_m.wait()
        cp_w.wait()
        lane = lax.iota(jnp.int32, LANES)
        row_idx = (base + lane) * L + pos_v[...]
        mut = mut_v[...]
        wt = wt_v[...]
        copies = []
        for i in range(LANES):
            copies.append(pltpu.async_copy(
                logits_hbm.at[row_idx[i]], rows_v.at[i], sem))
        for cp in copies:
            cp.wait()
        mut_val = jnp.zeros((LANES,), jnp.float32)
        wt_val = jnp.zeros((LANES,), jnp.float32)
        for i in range(LANES):
            lane_is_i = lane == i
            c_a = rows_v[i, pl.ds(0, LANES)]
            c_b = rows_v[i, pl.ds(LANES, LANES)]
            c_c = rows_v[i, pl.ds(V - LANES, LANES)]
            mut_val = _pick(c_a, mut, lane_is_i, 0, 16, 0, mut_val)
            mut_val = _pick(c_b, mut, lane_is_i, 16, 32, 16, mut_val)
            mut_val = _pick(c_c, mut, lane_is_i, 32, 33, V - LANES, mut_val)
            wt_val = _pick(c_a, wt, lane_is_i, 0, 16, 0, wt_val)
            wt_val = _pick(c_b, wt, lane_is_i, 16, 32, 16, wt_val)
            wt_val = _pick(c_c, wt, lane_is_i, 32, 33, V - LANES, wt_val)
        res_v[...] = mut_val / wt_val
        pltpu.sync_copy(res_v, out_hbm.at[pl.ds(base, LANES)])


def kernel(wt_logits, mutated_position_idx, mutant_token_idx, wt_token_idx):
    logits2d = wt_logits.reshape(B * L, V)
    pos = mutated_position_idx.astype(jnp.int32)
    mut = mutant_token_idx.astype(jnp.int32)
    wt = wt_token_idx.astype(jnp.int32)
    mesh = plsc.VectorSubcoreMesh(core_axis_name="c", subcore_axis_name="s")
    run = pl.kernel(
        _sc_body,
        out_type=jax.ShapeDtypeStruct((B,), jnp.float32),
        mesh=mesh,
        compiler_params=pltpu.CompilerParams(use_tc_tiling_on_sc=True),
        scratch_types=[
            pltpu.VMEM((LANES,), jnp.int32),
            pltpu.VMEM((LANES,), jnp.int32),
            pltpu.VMEM((LANES,), jnp.int32),
            pltpu.VMEM((LANES, V), jnp.float32),
            pltpu.VMEM((LANES,), jnp.float32),
            pltpu.SemaphoreType.DMA,
        ],
    )
    return run(logits2d, pos, mut, wt)


# speedup vs baseline: 9.1672x; 4.1177x over previous
import jax
import jax.numpy as jnp
from jax import lax
from jax.experimental import pallas as pl
from jax.experimental.pallas import tpu as pltpu
from jax.experimental.pallas import tpu_sc as plsc

B, L, V = 128, 2048, 33

_INFO = plsc.get_sparse_core_info()
NC = _INFO.num_cores
NS = _INFO.num_subcores
LANES = _INFO.num_lanes
NW_ACTIVE = B // LANES

_PLANE = B * L
_TB = 8 * L
_TL = 8 * 128
_SB = 128


def _sc_body(flat_hbm, pos_hbm, mut_hbm, wt_hbm, out_hbm,
             pos_v, mut_v, wt_v, idx_v, gat_v, res_v, sem):
    wid = lax.axis_index("s") * NC + lax.axis_index("c")

    @pl.when(wid < NW_ACTIVE)
    def _():
        base = wid * LANES
        cp_p = pltpu.async_copy(pos_hbm.at[pl.ds(base, LANES)], pos_v, sem)
        cp_m = pltpu.async_copy(mut_hbm.at[pl.ds(base, LANES)], mut_v, sem)
        cp_w = pltpu.async_copy(wt_hbm.at[pl.ds(base, LANES)], wt_v, sem)
        cp_p.wait()
        cp_m.wait()
        cp_w.wait()
        lane = lax.iota(jnp.int32, LANES)
        b = base + lane
        pos = pos_v[...]
        common = ((b >> 3) * _TB + (b & 7) * _SB
                  + (pos >> 7) * _TL + (pos & 127))
        idx_v[pl.ds(0, LANES)] = mut_v[...] * _PLANE + common
        idx_v[pl.ds(LANES, LANES)] = wt_v[...] * _PLANE + common
        pltpu.async_copy(flat_hbm.at[idx_v], gat_v, sem).wait()
        res_v[...] = gat_v[pl.ds(0, LANES)] / gat_v[pl.ds(LANES, LANES)]
        pltpu.sync_copy(res_v, out_hbm.at[pl.ds(base, LANES)])


def kernel(wt_logits, mutated_position_idx, mutant_token_idx, wt_token_idx):
    phys = (wt_logits.reshape(B // 8, 8, L // 128, 128, V)
            .transpose(4, 0, 2, 1, 3).reshape(-1))
    pos = mutated_position_idx.astype(jnp.int32)
    mut = mutant_token_idx.astype(jnp.int32)
    wt = wt_token_idx.astype(jnp.int32)
    mesh = plsc.VectorSubcoreMesh(core_axis_name="c", subcore_axis_name="s")
    run = pl.kernel(
        _sc_body,
        out_type=jax.ShapeDtypeStruct((B,), jnp.float32),
        mesh=mesh,
        scratch_types=[
            pltpu.VMEM((LANES,), jnp.int32),
            pltpu.VMEM((LANES,), jnp.int32),
            pltpu.VMEM((LANES,), jnp.int32),
            pltpu.VMEM((2 * LANES,), jnp.int32),
            pltpu.VMEM((2 * LANES,), jnp.float32),
            pltpu.VMEM((LANES,), jnp.float32),
            pltpu.SemaphoreType.DMA,
        ],
    )
    return run(phys, pos, mut, wt)


# speedup vs baseline: 9.9166x; 1.0817x over previous
import jax
import jax.numpy as jnp
from jax import lax
from jax.experimental import pallas as pl
from jax.experimental.pallas import tpu as pltpu
from jax.experimental.pallas import tpu_sc as plsc

B, L, V = 128, 2048, 33

_INFO = plsc.get_sparse_core_info()
NC = _INFO.num_cores
NS = _INFO.num_subcores
LANES = _INFO.num_lanes
NW_ACTIVE = B // LANES

_PLANE = B * L
_TB = 8 * L
_TL = 8 * 128
_SB = 128


MESH_CORES = 1


def _sc_body(flat_hbm, pos_hbm, mut_hbm, wt_hbm, out_hbm,
             pos_v, mut_v, wt_v, idx_v, gat_v, res_v, sem):
    wid = lax.axis_index("s") * MESH_CORES + lax.axis_index("c")

    @pl.when(wid < NW_ACTIVE)
    def _():
        base = wid * LANES
        cp_p = pltpu.async_copy(pos_hbm.at[pl.ds(base, LANES)], pos_v, sem)
        cp_m = pltpu.async_copy(mut_hbm.at[pl.ds(base, LANES)], mut_v, sem)
        cp_w = pltpu.async_copy(wt_hbm.at[pl.ds(base, LANES)], wt_v, sem)
        cp_p.wait()
        cp_m.wait()
        cp_w.wait()
        lane = lax.iota(jnp.int32, LANES)
        b = base + lane
        pos = pos_v[...]
        common = ((b >> 3) * _TB + (b & 7) * _SB
                  + (pos >> 7) * _TL + (pos & 127))
        idx_v[pl.ds(0, LANES)] = mut_v[...] * _PLANE + common
        idx_v[pl.ds(LANES, LANES)] = wt_v[...] * _PLANE + common
        pltpu.async_copy(flat_hbm.at[idx_v], gat_v, sem).wait()
        res_v[...] = gat_v[pl.ds(0, LANES)] / gat_v[pl.ds(LANES, LANES)]
        pltpu.sync_copy(res_v, out_hbm.at[pl.ds(base, LANES)])


def kernel(wt_logits, mutated_position_idx, mutant_token_idx, wt_token_idx):
    phys = (wt_logits.reshape(B // 8, 8, L // 128, 128, V)
            .transpose(4, 0, 2, 1, 3).reshape(-1))
    pos = mutated_position_idx.astype(jnp.int32)
    mut = mutant_token_idx.astype(jnp.int32)
    wt = wt_token_idx.astype(jnp.int32)
    mesh = plsc.VectorSubcoreMesh(
        core_axis_name="c", subcore_axis_name="s", num_cores=MESH_CORES)
    run = pl.kernel(
        _sc_body,
        out_type=jax.ShapeDtypeStruct((B,), jnp.float32),
        mesh=mesh,
        compiler_params=pltpu.CompilerParams(skip_device_barrier=True),
        scratch_types=[
            pltpu.VMEM((LANES,), jnp.int32),
            pltpu.VMEM((LANES,), jnp.int32),
            pltpu.VMEM((LANES,), jnp.int32),
            pltpu.VMEM((2 * LANES,), jnp.int32),
            pltpu.VMEM((2 * LANES,), jnp.float32),
            pltpu.VMEM((LANES,), jnp.float32),
            pltpu.SemaphoreType.DMA,
        ],
    )
    return run(phys, pos, mut, wt)


# speedup vs baseline: 28.4645x; 2.8704x over previous
import jax
import jax.numpy as jnp
from jax import lax
from jax.experimental import pallas as pl
from jax.experimental.pallas import tpu as pltpu

B, L, V = 128, 2048, 33
NT = B * 128


def _tc_body(xt_hbm, pos_s, mut_s, wt_s, rem_v, out_ref, mbuf, wbuf, sem):
    copies = []
    for b in range(B):
        p = pos_s[b]
        tl = p >> 7
        tb = b >> 3
        dst = pl.ds(128 * b, 128)
        copies.append(pltpu.make_async_copy(
            xt_hbm.at[mut_s[b], tb, tl], mbuf.at[:, dst], sem))
        copies.append(pltpu.make_async_copy(
            xt_hbm.at[wt_s[b], tb, tl], wbuf.at[:, dst], sem))
    for cp in copies:
        cp.start()
    for cp in copies:
        cp.wait()
    lane = lax.broadcasted_iota(jnp.int32, (1, NT), 1)
    subsel = (lane >> 7) & 7
    m1 = jnp.take_along_axis(mbuf[...], subsel, axis=0).reshape(B, 128)
    w1 = jnp.take_along_axis(wbuf[...], subsel, axis=0).reshape(B, 128)
    r = m1 / w1
    rem = jnp.broadcast_to(rem_v[...], (B, 128))
    out_ref[...] = jnp.take_along_axis(r, rem, axis=1)[:, 0:1]


def kernel(wt_logits, mutated_position_idx, mutant_token_idx, wt_token_idx):
    xt5 = (wt_logits.reshape(B // 8, 8, L // 128, 128, V)
           .transpose(4, 0, 2, 1, 3))
    pos = mutated_position_idx.astype(jnp.int32)
    mut = mutant_token_idx.astype(jnp.int32)
    wt = wt_token_idx.astype(jnp.int32)
    rem = (pos & 127).reshape(B, 1)
    out = pl.pallas_call(
        _tc_body,
        out_shape=jax.ShapeDtypeStruct((B, 1), jnp.float32),
        in_specs=[
            pl.BlockSpec(memory_space=pltpu.MemorySpace.HBM),
            pl.BlockSpec(memory_space=pltpu.MemorySpace.SMEM),
            pl.BlockSpec(memory_space=pltpu.MemorySpace.SMEM),
            pl.BlockSpec(memory_space=pltpu.MemorySpace.SMEM),
            pl.BlockSpec(memory_space=pltpu.MemorySpace.VMEM),
        ],
        out_specs=pl.BlockSpec(memory_space=pltpu.MemorySpace.VMEM),
        scratch_shapes=[
            pltpu.VMEM((8, NT), jnp.float32),
            pltpu.VMEM((8, NT), jnp.float32),
            pltpu.SemaphoreType.DMA,
        ],
    )(xt5, pos, mut, wt, rem)
    return out.reshape(B)


# speedup vs baseline: 39.7873x; 1.3978x over previous
import jax
import jax.numpy as jnp
from jax import lax
from jax.experimental import pallas as pl
from jax.experimental.pallas import tpu as pltpu

B, L, V = 128, 2048, 33
NT = B * 128


def _tc_body(xt_hbm, pos_s, mut_s, wt_s, rem_v, out_ref, mbuf, wbuf, sem):
    copies = []
    for b in range(B):
        p = pos_s[b]
        tl = p >> 7
        tb = b >> 3
        dst = pl.ds(128 * b, 128)
        copies.append(pltpu.make_async_copy(
            xt_hbm.at[mut_s[b], tb, tl], mbuf.at[:, dst], sem))
        copies.append(pltpu.make_async_copy(
            xt_hbm.at[wt_s[b], tb, tl], wbuf.at[:, dst], sem))
    for cp in copies:
        cp.start()
    for cp in copies:
        cp.wait()
    lane = lax.broadcasted_iota(jnp.int32, (1, NT), 1)
    subsel = (lane >> 7) & 7
    m1 = jnp.take_along_axis(mbuf[...], subsel, axis=0).reshape(B, 128)
    w1 = jnp.take_along_axis(wbuf[...], subsel, axis=0).reshape(B, 128)
    r = m1 / w1
    rem = jnp.broadcast_to(rem_v[...].reshape(B, 1), (B, 128))
    col = jnp.take_along_axis(r, rem, axis=1)[:, 0:1]
    out_ref[...] = jnp.transpose(col, (1, 0))


def kernel(wt_logits, mutated_position_idx, mutant_token_idx, wt_token_idx):
    xt5 = (wt_logits.reshape(B // 8, 8, L // 128, 128, V)
           .transpose(4, 0, 2, 1, 3))
    pos = mutated_position_idx.astype(jnp.int32)
    mut = mutant_token_idx.astype(jnp.int32)
    wt = wt_token_idx.astype(jnp.int32)
    rem = (pos & 127).reshape(1, B)
    out = pl.pallas_call(
        _tc_body,
        out_shape=jax.ShapeDtypeStruct((1, B), jnp.float32),
        in_specs=[
            pl.BlockSpec(memory_space=pltpu.MemorySpace.HBM),
            pl.BlockSpec(memory_space=pltpu.MemorySpace.SMEM),
            pl.BlockSpec(memory_space=pltpu.MemorySpace.SMEM),
            pl.BlockSpec(memory_space=pltpu.MemorySpace.SMEM),
            pl.BlockSpec(memory_space=pltpu.MemorySpace.VMEM),
        ],
        out_specs=pl.BlockSpec(memory_space=pltpu.MemorySpace.VMEM),
        scratch_shapes=[
            pltpu.VMEM((8, NT), jnp.float32),
            pltpu.VMEM((8, NT), jnp.float32),
            pltpu.SemaphoreType.DMA,
        ],
    )(xt5, pos, mut, wt, rem)
    return out.reshape(B)
